# baseline (speedup 1.0000x reference)
import functools

import jax
import jax.numpy as jnp
from jax import lax
from jax.experimental import pallas as pl
from jax.experimental.pallas import tpu as pltpu
from jax.experimental.pallas import tpu_sc as plsc

N = 10000
E = 320000
D = 128
H = 128
C = 64
NPAD = 10240
BN = 1024

ECH = 1600
NCH = E // ECH
GP = ECH // 16
UNROLL = 2
RPT = 4
RNODE = NPAD // 32
FH = 64
EPT = E // 16
SB = 40
NSB = EPT // SB
ZROWS = NPAD // 16


def _matmul_T(W, xT):
    return lax.dot_general(W, xT, (((1,), (0,)), ((), ())),
                           preferred_element_type=jnp.float32,
                           precision=lax.Precision.HIGHEST)


def _transpose(x):
    return jnp.transpose(x, (1, 0))


def _tc_pre_body(xT_ref, Ws_ref, qT_ref, qrows_ref):
    qT = _matmul_T(Ws_ref[...], xT_ref[...])
    qT_ref[...] = qT
    qrows_ref[0] = _transpose(qT[:FH, :])
    qrows_ref[1] = _transpose(qT[FH:, :])


def _tc_pre(xT, Ws):
    return pl.pallas_call(
        _tc_pre_body,
        grid=(NPAD // BN,),
        in_specs=[
            pl.BlockSpec((128, BN), lambda i: (0, i)),
            pl.BlockSpec((128, 128), lambda i: (0, 0)),
        ],
        out_specs=[
            pl.BlockSpec((128, BN), lambda i: (0, i)),
            pl.BlockSpec((2, BN, FH), lambda i: (0, i, 0)),
        ],
        out_shape=[
            jax.ShapeDtypeStruct((128, NPAD), jnp.float32),
            jax.ShapeDtypeStruct((2, NPAD, FH), jnp.float32),
        ],
    )(xT, Ws)


def _post_block(hT, Shalves, M, cnt, Wd, Wx, Wm, Wmx, Ws2, linW,
                pre_b, post_b, lin_b):
    S = _transpose(jnp.concatenate([Shalves[0], Shalves[1]], axis=-1))
    aT = _matmul_T(Wd, hT) + pre_b
    inv = 1.0 / jnp.maximum(cnt, 1.0)
    meanT = aT + S * inv
    ssumT = cnt * aT + S
    smaxT = jnp.where(cnt > 0, aT + M, 0.0)
    postT = (_matmul_T(Wx, hT) + _matmul_T(Wm, meanT)
             + _matmul_T(Wmx, smaxT) + _matmul_T(Ws2, ssumT) + post_b)
    return jax.nn.relu(_matmul_T(linW, postT) + lin_b)


def _tc_mid_body(hT_ref, S_ref, M_ref, cnt_ref, Wstk_ref, bstk_ref,
                 h1T_ref, qT2_ref, q2rows_ref):
    Wd, Wx, Wm, Wmx, Ws2, linW, Wsnext = (Wstk_ref[i] for i in range(7))
    pre_b, post_b, lin_b = (bstk_ref[i][:, None] for i in range(3))
    h1T = _post_block(hT_ref[...], S_ref[...], M_ref[...], cnt_ref[...],
                      Wd, Wx, Wm, Wmx, Ws2, linW, pre_b, post_b, lin_b)
    h1T_ref[...] = h1T
    qT2 = _matmul_T(Wsnext, h1T)
    qT2_ref[...] = qT2
    q2rows_ref[0] = _transpose(qT2[:FH, :])
    q2rows_ref[1] = _transpose(qT2[FH:, :])


def _tc_mid(hT, Spair, M, cnt2d, Wstk, bstk):
    return pl.pallas_call(
        _tc_mid_body,
        grid=(NPAD // BN,),
        in_specs=[
            pl.BlockSpec((128, BN), lambda i: (0, i)),
            pl.BlockSpec((2, BN, FH), lambda i: (0, i, 0)),
            pl.BlockSpec((128, BN), lambda i: (0, i)),
            pl.BlockSpec((1, BN), lambda i: (0, i)),
            pl.BlockSpec((7, 128, 128), lambda i: (0, 0, 0)),
            pl.BlockSpec((3, 128), lambda i: (0, 0)),
        ],
        out_specs=[
            pl.BlockSpec((128, BN), lambda i: (0, i)),
            pl.BlockSpec((128, BN), lambda i: (0, i)),
            pl.BlockSpec((2, BN, FH), lambda i: (0, i, 0)),
        ],
        out_shape=[
            jax.ShapeDtypeStruct((128, NPAD), jnp.float32),
            jax.ShapeDtypeStruct((128, NPAD), jnp.float32),
            jax.ShapeDtypeStruct((2, NPAD, FH), jnp.float32),
        ],
    )(hT, Spair, M, cnt2d, Wstk, bstk)


def _tc_fin_body(hT_ref, S_ref, M_ref, cnt_ref, Wstk_ref, bstk_ref,
                 Wout_ref, zT_ref):
    Wd, Wx, Wm, Wmx, Ws2, linW = (Wstk_ref[i] for i in range(6))
    pre_b, post_b, lin_b = (bstk_ref[i][:, None] for i in range(3))
    h2T = _post_block(hT_ref[...], S_ref[...], M_ref[...], cnt_ref[...],
                      Wd, Wx, Wm, Wmx, Ws2, linW, pre_b, post_b, lin_b)
    zT_ref[...] = lax.dot_general(Wout_ref[...], h2T, (((1,), (0,)), ((), ())),
                                  preferred_element_type=jnp.float32,
                                  precision=lax.Precision.HIGHEST)


def _tc_fin(hT, Spair, M, cnt2d, Wstk, bstk, Wout):
    return pl.pallas_call(
        _tc_fin_body,
        grid=(NPAD // BN,),
        in_specs=[
            pl.BlockSpec((128, BN), lambda i: (0, i)),
            pl.BlockSpec((2, BN, FH), lambda i: (0, i, 0)),
            pl.BlockSpec((128, BN), lambda i: (0, i)),
            pl.BlockSpec((1, BN), lambda i: (0, i)),
            pl.BlockSpec((6, 128, 128), lambda i: (0, 0, 0)),
            pl.BlockSpec((3, 128), lambda i: (0, 0)),
            pl.BlockSpec((C, 128), lambda i: (0, 0)),
        ],
        out_specs=pl.BlockSpec((C, BN), lambda i: (0, i)),
        out_shape=jax.ShapeDtypeStruct((C, NPAD), jnp.float32),
    )(hT, Spair, M, cnt2d, Wstk, bstk, Wout)


def _make_sc_body(with_cnt):
    def _sc_body(src_hbm, dst_hbm, qT_hbm, qrows_hbm, zeros_hbm,
                 S_hbm, M_hbm, cnt_hbm,
                 q_v, m0, m1v, m2v, m3, cnt_v, src_v, dst_v,
                 sidx_v, didx_v, grows_v, acc_spm):
        maxs = [m0, m1v, m2v, m3]
        c = lax.axis_index("c")
        s = lax.axis_index("s")
        wid = s * 2 + c
        rbase = wid * RPT
        nbase = wid * RNODE

        pltpu.sync_copy(qT_hbm.at[pl.ds(rbase, RPT), :], q_v)
        pltpu.sync_copy(zeros_hbm, acc_spm.at[pl.ds(s * ZROWS, ZROWS), :])
        qhalf_hbm = qrows_hbm.at[c]

        zeros16 = jnp.zeros((16,), jnp.float32)
        ninf16 = jnp.full((16,), -3.0e38, jnp.float32)
        iota16 = lax.iota(jnp.int32, 16)
        ones16 = jnp.ones((16,), jnp.float32)
        false16 = jnp.zeros((16,), jnp.bool_)
        rvecs = [jnp.full((16,), r, jnp.int32) for r in range(RPT)]

        def zbody(i, carry):
            for r in range(RPT):
                maxs[r][pl.ds(i * 16, 16)] = ninf16
            return carry
        lax.fori_loop(0, NPAD // 16, zbody, 0)

        if with_cnt:
            def cbody(i, carry):
                cnt_v[pl.ds(i * 16, 16)] = zeros16
                return carry
            lax.fori_loop(0, RNODE // 16, cbody, 0)

        plsc.subcore_barrier()

        eb = s * EPT

        def sum_batch(k, carry):
            pltpu.sync_copy(src_hbm.at[pl.ds(eb + k * SB, SB)], sidx_v)
            pltpu.sync_copy(dst_hbm.at[pl.ds(eb + k * SB, SB)], didx_v)
            pltpu.sync_copy(qhalf_hbm.at[sidx_v], grows_v)
            pltpu.sync_copy(grows_v, acc_spm.at[didx_v], add=True)
            return carry
        lax.fori_loop(0, NSB, sum_batch, 0)

        def do_group(g, ovf):
            s16 = src_v[pl.ds(g * 16, 16)]
            d16 = dst_v[pl.ds(g * 16, 16)]
            occ, _last = plsc.scan_count(d16)
            m1 = occ == 1
            m2 = occ == 2
            ovf = ovf | (occ >= 3)
            if with_cnt:
                mine = (d16 >= nbase) & (d16 < nbase + RNODE)
                plsc.addupdate_scatter(cnt_v, [d16 - nbase], ones16, mask=mine)
            vs = []
            for r in range(RPT):
                v = plsc.load_gather(q_v, [rvecs[r], s16])
                vs.append(v)
                cur = plsc.load_gather(maxs[r], [d16])
                plsc.store_scatter(maxs[r], [d16],
                                   jnp.maximum(cur, v), mask=m1)

            @pl.when(jnp.any(occ >= 2))
            def _round2():
                for r in range(RPT):
                    cur2 = plsc.load_gather(maxs[r], [d16], mask=m2)
                    plsc.store_scatter(maxs[r], [d16],
                                       jnp.maximum(cur2, vs[r]), mask=m2)
            return ovf

        def chunk_body(ch, carry):
            pltpu.sync_copy(src_hbm.at[pl.ds(ch * ECH, ECH)], src_v)
            pltpu.sync_copy(dst_hbm.at[pl.ds(ch * ECH, ECH)], dst_v)

            def group_body(i, ovf):
                for u in range(UNROLL):
                    ovf = do_group(i * UNROLL + u, ovf)
                return ovf
            ovf = lax.fori_loop(0, GP // UNROLL, group_body, false16)

            @pl.when(jnp.any(ovf))
            def _fixup():
                def fg(g, carry2):
                    s16 = src_v[pl.ds(g * 16, 16)]
                    d16 = dst_v[pl.ds(g * 16, 16)]
                    for r in range(RPT):
                        v = plsc.load_gather(q_v, [rvecs[r], s16])
                        def fl(l, c3, v=v, d16=d16, r=r):
                            lm = iota16 == l
                            cur = plsc.load_gather(maxs[r], [d16], mask=lm)
                            plsc.store_scatter(maxs[r], [d16],
                                               jnp.maximum(cur, v), mask=lm)
                            return c3
                        lax.fori_loop(0, 16, fl, 0)
                    return carry2
                lax.fori_loop(0, GP, fg, 0)
            return carry
        lax.fori_loop(0, NCH, chunk_body, 0)

        plsc.subcore_barrier()
        for r in range(RPT):
            pltpu.sync_copy(maxs[r], M_hbm.at[rbase + r, :])
        pltpu.sync_copy(acc_spm.at[pl.ds(s * ZROWS, ZROWS), :],
                        S_hbm.at[c].at[pl.ds(s * ZROWS, ZROWS), :])
        if with_cnt:
            pltpu.sync_copy(cnt_v, cnt_hbm.at[pl.ds(nbase, RNODE)])
    return _sc_body


@functools.partial(jax.jit, static_argnames=("with_cnt",))
def _sc_segment(src, dst, qT, qrows, zeros, with_cnt=True):
    mesh = plsc.VectorSubcoreMesh(core_axis_name="c", subcore_axis_name="s")
    f = pl.kernel(
        _make_sc_body(with_cnt),
        mesh=mesh,
        compiler_params=pltpu.CompilerParams(needs_layout_passes=False, use_tc_tiling_on_sc=False),
        out_type=[
            jax.ShapeDtypeStruct((2, NPAD, FH), jnp.float32),
            jax.ShapeDtypeStruct((128, NPAD), jnp.float32),
            jax.ShapeDtypeStruct((NPAD,), jnp.float32),
        ],
        scratch_types=(
            [pltpu.VMEM((RPT, NPAD), jnp.float32)]
            + [pltpu.VMEM((NPAD,), jnp.float32) for _ in range(4)]
            + [
                pltpu.VMEM((RNODE,), jnp.float32),
                pltpu.VMEM((ECH,), jnp.int32),
                pltpu.VMEM((ECH,), jnp.int32),
                pltpu.VMEM((SB,), jnp.int32),
                pltpu.VMEM((SB,), jnp.int32),
                pltpu.VMEM((SB, FH), jnp.float32),
                pltpu.VMEM_SHARED((NPAD, FH), jnp.float32),
            ]
        ),
    )
    return f(src, dst, qT, qrows, zeros)


def kernel(x, edge_index, pre_W1, pre_b1, post_W1, post_b1, lin_W1, lin_b1,
           pre_W2, pre_b2, post_W2, post_b2, lin_W2, lin_b2, W_out, b_out):
    src = edge_index[0]
    dst = edge_index[1]
    xT = jnp.zeros((128, NPAD), jnp.float32).at[:, :N].set(x.T)
    zeros = jnp.zeros((ZROWS, FH), jnp.float32)

    Wd1, Ws1 = pre_W1[:, :D], pre_W1[:, D:]
    Wx1, Wm1, Wmx1, Ws21 = (post_W1[:, 0:128], post_W1[:, 128:256],
                            post_W1[:, 256:384], post_W1[:, 384:512])
    Wd2, Ws2_ = pre_W2[:, :H], pre_W2[:, H:]
    Wx2, Wm2, Wmx2, Ws22 = (post_W2[:, 0:128], post_W2[:, 128:256],
                            post_W2[:, 256:384], post_W2[:, 384:512])

    Wstk1 = jnp.stack([Wd1, Wx1, Wm1, Wmx1, Ws21, lin_W1, Ws2_])
    bstk1 = jnp.stack([pre_b1, post_b1, lin_b1])
    Wstk2 = jnp.stack([Wd2, Wx2, Wm2, Wmx2, Ws22, lin_W2])
    bstk2 = jnp.stack([pre_b2, post_b2, lin_b2])

    qT1, q1rows = _tc_pre(xT, Ws1)
    S1, M1, cnt = _sc_segment(src, dst, qT1, q1rows, zeros)
    cnt2d = cnt.reshape(1, NPAD)
    h1T, qT2, q2rows = _tc_mid(xT, S1, M1, cnt2d, Wstk1, bstk1)
    S2, M2, _ = _sc_segment(src, dst, qT2, q2rows, zeros, with_cnt=False)
    zT = _tc_fin(h1T, S2, M2, cnt2d, Wstk2, bstk2, W_out)
    return zT[:, :N].T + b_out


# speedup vs baseline: 1.4762x; 1.4762x over previous
import functools

import jax
import jax.numpy as jnp
from jax import lax
from jax.experimental import pallas as pl
from jax.experimental.pallas import tpu as pltpu
from jax.experimental.pallas import tpu_sc as plsc

N = 10000
E = 320000
D = 128
H = 128
C = 64
NPAD = 10240
BN = 1024

ECH = 1280
NCH = E // ECH
GP = ECH // 16
UNROLL = 2
RPT = 4
RNODE = NPAD // 32
FH = 64
SSH = ECH // 16
ZROWS = NPAD // 16


def _matmul_T(W, xT):
    return lax.dot_general(W, xT, (((1,), (0,)), ((), ())),
                           preferred_element_type=jnp.float32,
                           precision=lax.Precision.HIGHEST)


def _transpose(x):
    return jnp.transpose(x, (1, 0))


def _tc_pre_body(xT_ref, Ws_ref, qT_ref, qrows_ref):
    qT = _matmul_T(Ws_ref[...], xT_ref[...])
    qT_ref[...] = qT
    qrows_ref[0] = _transpose(qT[:FH, :])
    qrows_ref[1] = _transpose(qT[FH:, :])


def _tc_pre(xT, Ws):
    return pl.pallas_call(
        _tc_pre_body,
        grid=(NPAD // BN,),
        in_specs=[
            pl.BlockSpec((128, BN), lambda i: (0, i)),
            pl.BlockSpec((128, 128), lambda i: (0, 0)),
        ],
        out_specs=[
            pl.BlockSpec((128, BN), lambda i: (0, i)),
            pl.BlockSpec((2, BN, FH), lambda i: (0, i, 0)),
        ],
        out_shape=[
            jax.ShapeDtypeStruct((128, NPAD), jnp.float32),
            jax.ShapeDtypeStruct((2, NPAD, FH), jnp.float32),
        ],
    )(xT, Ws)


def _post_block(hT, Shalves, M, cnt, Wd, Wx, Wm, Wmx, Ws2, linW,
                pre_b, post_b, lin_b):
    S = _transpose(jnp.concatenate([Shalves[0], Shalves[1]], axis=-1))
    aT = _matmul_T(Wd, hT) + pre_b
    inv = 1.0 / jnp.maximum(cnt, 1.0)
    meanT = aT + S * inv
    ssumT = cnt * aT + S
    smaxT = jnp.where(cnt > 0, aT + M, 0.0)
    postT = (_matmul_T(Wx, hT) + _matmul_T(Wm, meanT)
             + _matmul_T(Wmx, smaxT) + _matmul_T(Ws2, ssumT) + post_b)
    return jax.nn.relu(_matmul_T(linW, postT) + lin_b)


def _tc_mid_body(hT_ref, S_ref, M_ref, cnt_ref, Wstk_ref, bstk_ref,
                 h1T_ref, qT2_ref, q2rows_ref):
    Wd, Wx, Wm, Wmx, Ws2, linW, Wsnext = (Wstk_ref[i] for i in range(7))
    pre_b, post_b, lin_b = (bstk_ref[i][:, None] for i in range(3))
    h1T = _post_block(hT_ref[...], S_ref[...], M_ref[...], cnt_ref[...],
                      Wd, Wx, Wm, Wmx, Ws2, linW, pre_b, post_b, lin_b)
    h1T_ref[...] = h1T
    qT2 = _matmul_T(Wsnext, h1T)
    qT2_ref[...] = qT2
    q2rows_ref[0] = _transpose(qT2[:FH, :])
    q2rows_ref[1] = _transpose(qT2[FH:, :])


def _tc_mid(hT, Spair, M, cnt2d, Wstk, bstk):
    return pl.pallas_call(
        _tc_mid_body,
        grid=(NPAD // BN,),
        in_specs=[
            pl.BlockSpec((128, BN), lambda i: (0, i)),
            pl.BlockSpec((2, BN, FH), lambda i: (0, i, 0)),
            pl.BlockSpec((128, BN), lambda i: (0, i)),
            pl.BlockSpec((1, BN), lambda i: (0, i)),
            pl.BlockSpec((7, 128, 128), lambda i: (0, 0, 0)),
            pl.BlockSpec((3, 128), lambda i: (0, 0)),
        ],
        out_specs=[
            pl.BlockSpec((128, BN), lambda i: (0, i)),
            pl.BlockSpec((128, BN), lambda i: (0, i)),
            pl.BlockSpec((2, BN, FH), lambda i: (0, i, 0)),
        ],
        out_shape=[
            jax.ShapeDtypeStruct((128, NPAD), jnp.float32),
            jax.ShapeDtypeStruct((128, NPAD), jnp.float32),
            jax.ShapeDtypeStruct((2, NPAD, FH), jnp.float32),
        ],
    )(hT, Spair, M, cnt2d, Wstk, bstk)


def _tc_fin_body(hT_ref, S_ref, M_ref, cnt_ref, Wstk_ref, bstk_ref,
                 Wout_ref, zT_ref):
    Wd, Wx, Wm, Wmx, Ws2, linW = (Wstk_ref[i] for i in range(6))
    pre_b, post_b, lin_b = (bstk_ref[i][:, None] for i in range(3))
    h2T = _post_block(hT_ref[...], S_ref[...], M_ref[...], cnt_ref[...],
                      Wd, Wx, Wm, Wmx, Ws2, linW, pre_b, post_b, lin_b)
    zT_ref[...] = lax.dot_general(Wout_ref[...], h2T, (((1,), (0,)), ((), ())),
                                  preferred_element_type=jnp.float32,
                                  precision=lax.Precision.HIGHEST)


def _tc_fin(hT, Spair, M, cnt2d, Wstk, bstk, Wout):
    return pl.pallas_call(
        _tc_fin_body,
        grid=(NPAD // BN,),
        in_specs=[
            pl.BlockSpec((128, BN), lambda i: (0, i)),
            pl.BlockSpec((2, BN, FH), lambda i: (0, i, 0)),
            pl.BlockSpec((128, BN), lambda i: (0, i)),
            pl.BlockSpec((1, BN), lambda i: (0, i)),
            pl.BlockSpec((6, 128, 128), lambda i: (0, 0, 0)),
            pl.BlockSpec((3, 128), lambda i: (0, 0)),
            pl.BlockSpec((C, 128), lambda i: (0, 0)),
        ],
        out_specs=pl.BlockSpec((C, BN), lambda i: (0, i)),
        out_shape=jax.ShapeDtypeStruct((C, NPAD), jnp.float32),
    )(hT, Spair, M, cnt2d, Wstk, bstk, Wout)


def _make_sc_body(with_cnt):
    def _sc_body(src_hbm, dst_hbm, qT_hbm, qrows_hbm, zeros_hbm,
                 S_hbm, M_hbm, cnt_hbm,
                 q_v, m0, m1v, m2v, m3, cnt_v, src_v, dst_v,
                 grows_v, gsem, acc_spm):
        maxs = [m0, m1v, m2v, m3]
        c = lax.axis_index("c")
        s = lax.axis_index("s")
        wid = s * 2 + c
        rbase = wid * RPT
        nbase = wid * RNODE

        pltpu.sync_copy(qT_hbm.at[pl.ds(rbase, RPT), :], q_v)
        pltpu.sync_copy(zeros_hbm, acc_spm.at[pl.ds(s * ZROWS, ZROWS), :])
        qhalf_hbm = qrows_hbm.at[c]

        zeros16 = jnp.zeros((16,), jnp.float32)
        ninf16 = jnp.full((16,), -3.0e38, jnp.float32)
        iota16 = lax.iota(jnp.int32, 16)
        ones16 = jnp.ones((16,), jnp.float32)
        false16 = jnp.zeros((16,), jnp.bool_)
        rvecs = [jnp.full((16,), r, jnp.int32) for r in range(RPT)]

        def zbody(i, carry):
            for r in range(RPT):
                maxs[r][pl.ds(i * 16, 16)] = ninf16
            return carry
        lax.fori_loop(0, NPAD // 16, zbody, 0)

        if with_cnt:
            def cbody(i, carry):
                cnt_v[pl.ds(i * 16, 16)] = zeros16
                return carry
            lax.fori_loop(0, RNODE // 16, cbody, 0)

        plsc.subcore_barrier()

        def do_group(g, ovf):
            s16 = src_v[pl.ds(g * 16, 16)]
            d16 = dst_v[pl.ds(g * 16, 16)]
            occ, _last = plsc.scan_count(d16)
            m1 = occ == 1
            m2 = occ == 2
            ovf = ovf | (occ >= 3)
            if with_cnt:
                mine = (d16 >= nbase) & (d16 < nbase + RNODE)
                plsc.addupdate_scatter(cnt_v, [d16 - nbase], ones16, mask=mine)
            vs = []
            for r in range(RPT):
                v = plsc.load_gather(q_v, [rvecs[r], s16])
                vs.append(v)
                cur = plsc.load_gather(maxs[r], [d16])
                plsc.store_scatter(maxs[r], [d16],
                                   jnp.maximum(cur, v), mask=m1)

            @pl.when(jnp.any(occ >= 2))
            def _round2():
                for r in range(RPT):
                    cur2 = plsc.load_gather(maxs[r], [d16], mask=m2)
                    plsc.store_scatter(maxs[r], [d16],
                                       jnp.maximum(cur2, vs[r]), mask=m2)
            return ovf

        def chunk_body(ch, carry):
            pltpu.sync_copy(src_hbm.at[pl.ds(ch * ECH, ECH)], src_v)
            pltpu.sync_copy(dst_hbm.at[pl.ds(ch * ECH, ECH)], dst_v)
            gd = pltpu.async_copy(
                qhalf_hbm.at[src_v.at[pl.ds(s * SSH, SSH)]], grows_v, gsem)

            def group_body(i, ovf):
                for u in range(UNROLL):
                    ovf = do_group(i * UNROLL + u, ovf)
                return ovf
            ovf = lax.fori_loop(0, GP // UNROLL, group_body, false16)

            @pl.when(jnp.any(ovf))
            def _fixup():
                def fg(g, carry2):
                    s16 = src_v[pl.ds(g * 16, 16)]
                    d16 = dst_v[pl.ds(g * 16, 16)]
                    for r in range(RPT):
                        v = plsc.load_gather(q_v, [rvecs[r], s16])
                        def fl(l, c3, v=v, d16=d16, r=r):
                            lm = iota16 == l
                            cur = plsc.load_gather(maxs[r], [d16], mask=lm)
                            plsc.store_scatter(maxs[r], [d16],
                                               jnp.maximum(cur, v), mask=lm)
                            return c3
                        lax.fori_loop(0, 16, fl, 0)
                    return carry2
                lax.fori_loop(0, GP, fg, 0)

            gd.wait()
            pltpu.sync_copy(grows_v,
                            acc_spm.at[dst_v.at[pl.ds(s * SSH, SSH)]],
                            add=True)
            return carry
        lax.fori_loop(0, NCH, chunk_body, 0)

        plsc.subcore_barrier()
        for r in range(RPT):
            pltpu.sync_copy(maxs[r], M_hbm.at[rbase + r, :])
        pltpu.sync_copy(acc_spm.at[pl.ds(s * ZROWS, ZROWS), :],
                        S_hbm.at[c].at[pl.ds(s * ZROWS, ZROWS), :])
        if with_cnt:
            pltpu.sync_copy(cnt_v, cnt_hbm.at[pl.ds(nbase, RNODE)])
    return _sc_body


@functools.partial(jax.jit, static_argnames=("with_cnt",))
def _sc_segment(src, dst, qT, qrows, zeros, with_cnt=True):
    mesh = plsc.VectorSubcoreMesh(core_axis_name="c", subcore_axis_name="s")
    f = pl.kernel(
        _make_sc_body(with_cnt),
        mesh=mesh,
        compiler_params=pltpu.CompilerParams(needs_layout_passes=False, use_tc_tiling_on_sc=False),
        out_type=[
            jax.ShapeDtypeStruct((2, NPAD, FH), jnp.float32),
            jax.ShapeDtypeStruct((128, NPAD), jnp.float32),
            jax.ShapeDtypeStruct((NPAD,), jnp.float32),
        ],
        scratch_types=(
            [pltpu.VMEM((RPT, NPAD), jnp.float32)]
            + [pltpu.VMEM((NPAD,), jnp.float32) for _ in range(4)]
            + [
                pltpu.VMEM((RNODE,), jnp.float32),
                pltpu.VMEM((ECH,), jnp.int32),
                pltpu.VMEM((ECH,), jnp.int32),
                pltpu.VMEM((SSH, FH), jnp.float32),
                pltpu.SemaphoreType.DMA,
                pltpu.VMEM_SHARED((NPAD, FH), jnp.float32),
            ]
        ),
    )
    return f(src, dst, qT, qrows, zeros)


def kernel(x, edge_index, pre_W1, pre_b1, post_W1, post_b1, lin_W1, lin_b1,
           pre_W2, pre_b2, post_W2, post_b2, lin_W2, lin_b2, W_out, b_out):
    src = edge_index[0]
    dst = edge_index[1]
    xT = jnp.zeros((128, NPAD), jnp.float32).at[:, :N].set(x.T)
    zeros = jnp.zeros((ZROWS, FH), jnp.float32)

    Wd1, Ws1 = pre_W1[:, :D], pre_W1[:, D:]
    Wx1, Wm1, Wmx1, Ws21 = (post_W1[:, 0:128], post_W1[:, 128:256],
                            post_W1[:, 256:384], post_W1[:, 384:512])
    Wd2, Ws2_ = pre_W2[:, :H], pre_W2[:, H:]
    Wx2, Wm2, Wmx2, Ws22 = (post_W2[:, 0:128], post_W2[:, 128:256],
                            post_W2[:, 256:384], post_W2[:, 384:512])

    Wstk1 = jnp.stack([Wd1, Wx1, Wm1, Wmx1, Ws21, lin_W1, Ws2_])
    bstk1 = jnp.stack([pre_b1, post_b1, lin_b1])
    Wstk2 = jnp.stack([Wd2, Wx2, Wm2, Wmx2, Ws22, lin_W2])
    bstk2 = jnp.stack([pre_b2, post_b2, lin_b2])

    qT1, q1rows = _tc_pre(xT, Ws1)
    S1, M1, cnt = _sc_segment(src, dst, qT1, q1rows, zeros)
    cnt2d = cnt.reshape(1, NPAD)
    h1T, qT2, q2rows = _tc_mid(xT, S1, M1, cnt2d, Wstk1, bstk1)
    S2, M2, _ = _sc_segment(src, dst, qT2, q2rows, zeros, with_cnt=False)
    zT = _tc_fin(h1T, S2, M2, cnt2d, Wstk2, bstk2, W_out)
    return zT[:, :N].T + b_out


# speedup vs baseline: 1.4790x; 1.0019x over previous
import functools

import jax
import jax.numpy as jnp
from jax import lax
from jax.experimental import pallas as pl
from jax.experimental.pallas import tpu as pltpu
from jax.experimental.pallas import tpu_sc as plsc

N = 10000
E = 320000
D = 128
H = 128
C = 64
NPAD = 10240
BN = 1024

ECH = 3200
NCH = E // ECH
GP = ECH // 16
UNROLL = 2
RPT = 4
RNODE = NPAD // 32


def _matmul_T(W, xT):
    return lax.dot_general(W, xT, (((1,), (0,)), ((), ())),
                           preferred_element_type=jnp.float32,
                           precision=lax.Precision.HIGHEST)


def _tc_pre_body(xT_ref, Ws_ref, qT_ref):
    qT_ref[...] = _matmul_T(Ws_ref[...], xT_ref[...])


def _tc_pre(xT, Ws):
    return pl.pallas_call(
        _tc_pre_body,
        grid=(NPAD // BN,),
        in_specs=[
            pl.BlockSpec((128, BN), lambda i: (0, i)),
            pl.BlockSpec((128, 128), lambda i: (0, 0)),
        ],
        out_specs=pl.BlockSpec((128, BN), lambda i: (0, i)),
        out_shape=jax.ShapeDtypeStruct((128, NPAD), jnp.float32),
    )(xT, Ws)


def _tc_mid_body(hT_ref, S_ref, M_ref, cnt_ref, Wstk_ref, bstk_ref, h1T_ref, qT2_ref):
    hT = hT_ref[...]
    S = S_ref[...]
    M = M_ref[...]
    cnt = cnt_ref[...]
    Wd, Wx, Wm, Wmx, Ws2, linW, Wsnext = (Wstk_ref[i] for i in range(7))
    pre_b, post_b, lin_b = (bstk_ref[i][:, None] for i in range(3))
    aT = _matmul_T(Wd, hT) + pre_b
    inv = 1.0 / jnp.maximum(cnt, 1.0)
    meanT = aT + S * inv
    ssumT = cnt * aT + S
    smaxT = jnp.where(cnt > 0, aT + M, 0.0)
    postT = (_matmul_T(Wx, hT) + _matmul_T(Wm, meanT)
             + _matmul_T(Wmx, smaxT) + _matmul_T(Ws2, ssumT) + post_b)
    h1T = jax.nn.relu(_matmul_T(linW, postT) + lin_b)
    h1T_ref[...] = h1T
    qT2_ref[...] = _matmul_T(Wsnext, h1T)


def _tc_mid(hT, S, M, cnt2d, Wstk, bstk):
    return pl.pallas_call(
        _tc_mid_body,
        grid=(NPAD // BN,),
        in_specs=[
            pl.BlockSpec((128, BN), lambda i: (0, i)),
            pl.BlockSpec((128, BN), lambda i: (0, i)),
            pl.BlockSpec((128, BN), lambda i: (0, i)),
            pl.BlockSpec((1, BN), lambda i: (0, i)),
            pl.BlockSpec((7, 128, 128), lambda i: (0, 0, 0)),
            pl.BlockSpec((3, 128), lambda i: (0, 0)),
        ],
        out_specs=[
            pl.BlockSpec((128, BN), lambda i: (0, i)),
            pl.BlockSpec((128, BN), lambda i: (0, i)),
        ],
        out_shape=[
            jax.ShapeDtypeStruct((128, NPAD), jnp.float32),
            jax.ShapeDtypeStruct((128, NPAD), jnp.float32),
        ],
    )(hT, S, M, cnt2d, Wstk, bstk)


def _tc_fin_body(hT_ref, S_ref, M_ref, cnt_ref, Wstk_ref, bstk_ref, Wout_ref, zT_ref):
    hT = hT_ref[...]
    S = S_ref[...]
    M = M_ref[...]
    cnt = cnt_ref[...]
    Wd, Wx, Wm, Wmx, Ws2, linW = (Wstk_ref[i] for i in range(6))
    pre_b, post_b, lin_b = (bstk_ref[i][:, None] for i in range(3))
    aT = _matmul_T(Wd, hT) + pre_b
    inv = 1.0 / jnp.maximum(cnt, 1.0)
    meanT = aT + S * inv
    ssumT = cnt * aT + S
    smaxT = jnp.where(cnt > 0, aT + M, 0.0)
    postT = (_matmul_T(Wx, hT) + _matmul_T(Wm, meanT)
             + _matmul_T(Wmx, smaxT) + _matmul_T(Ws2, ssumT) + post_b)
    h2T = jax.nn.relu(_matmul_T(linW, postT) + lin_b)
    zT_ref[...] = lax.dot_general(Wout_ref[...], h2T, (((1,), (0,)), ((), ())),
                                  preferred_element_type=jnp.float32,
                                  precision=lax.Precision.HIGHEST)


def _tc_fin(hT, S, M, cnt2d, Wstk, bstk, Wout):
    return pl.pallas_call(
        _tc_fin_body,
        grid=(NPAD // BN,),
        in_specs=[
            pl.BlockSpec((128, BN), lambda i: (0, i)),
            pl.BlockSpec((128, BN), lambda i: (0, i)),
            pl.BlockSpec((128, BN), lambda i: (0, i)),
            pl.BlockSpec((1, BN), lambda i: (0, i)),
            pl.BlockSpec((6, 128, 128), lambda i: (0, 0, 0)),
            pl.BlockSpec((3, 128), lambda i: (0, 0)),
            pl.BlockSpec((C, 128), lambda i: (0, 0)),
        ],
        out_specs=pl.BlockSpec((C, BN), lambda i: (0, i)),
        out_shape=jax.ShapeDtypeStruct((C, NPAD), jnp.float32),
    )(hT, S, M, cnt2d, Wstk, bstk, Wout)


def _make_sc_body(with_cnt):
    def _sc_body(src_hbm, dst_hbm, qT_hbm, S_hbm, M_hbm, cnt_hbm,
                 q_v, s0, s1, s2, s3, m0, m1v, m2v, m3, cnt_v, src_v, dst_v):
        sums = [s0, s1, s2, s3]
        maxs = [m0, m1v, m2v, m3]
        c = lax.axis_index("c")
        s = lax.axis_index("s")
        wid = s * 2 + c
        rbase = wid * RPT
        nbase = wid * RNODE

        pltpu.sync_copy(qT_hbm.at[pl.ds(rbase, RPT), :], q_v)

        zeros16 = jnp.zeros((16,), jnp.float32)
        ninf16 = jnp.full((16,), -3.0e38, jnp.float32)
        iota16 = lax.iota(jnp.int32, 16)
        ones16 = jnp.ones((16,), jnp.float32)
        false16 = jnp.zeros((16,), jnp.bool_)
        rvecs = [jnp.full((16,), r, jnp.int32) for r in range(RPT)]

        def zbody(i, carry):
            for r in range(RPT):
                sums[r][pl.ds(i * 16, 16)] = zeros16
                maxs[r][pl.ds(i * 16, 16)] = ninf16
            return carry
        lax.fori_loop(0, NPAD // 16, zbody, 0)

        if with_cnt:
            def cbody(i, carry):
                cnt_v[pl.ds(i * 16, 16)] = zeros16
                return carry
            lax.fori_loop(0, RNODE // 16, cbody, 0)

        def do_group(g, ovf):
            s16 = src_v[pl.ds(g * 16, 16)]
            d16 = dst_v[pl.ds(g * 16, 16)]
            occ, _last = plsc.scan_count(d16)
            m1 = occ == 1
            m2 = occ == 2
            ovf = ovf | (occ >= 3)
            if with_cnt:
                mine = (d16 >= nbase) & (d16 < nbase + RNODE)
                plsc.addupdate_scatter(cnt_v, [d16 - nbase], ones16, mask=mine)
            vs = []
            for r in range(RPT):
                v = plsc.load_gather(q_v, [rvecs[r], s16])
                vs.append(v)
                plsc.addupdate_scatter(sums[r], [d16], v)
                cur = plsc.load_gather(maxs[r], [d16])
                plsc.store_scatter(maxs[r], [d16],
                                   jnp.maximum(cur, v), mask=m1)

            @pl.when(jnp.any(occ >= 2))
            def _round2():
                for r in range(RPT):
                    cur2 = plsc.load_gather(maxs[r], [d16], mask=m2)
                    plsc.store_scatter(maxs[r], [d16],
                                       jnp.maximum(cur2, vs[r]), mask=m2)
            return ovf

        def chunk_body(ch, carry):
            pltpu.sync_copy(src_hbm.at[pl.ds(ch * ECH, ECH)], src_v)
            pltpu.sync_copy(dst_hbm.at[pl.ds(ch * ECH, ECH)], dst_v)

            def group_body(i, ovf):
                for u in range(UNROLL):
                    ovf = do_group(i * UNROLL + u, ovf)
                return ovf
            ovf = lax.fori_loop(0, GP // UNROLL, group_body, false16)

            @pl.when(jnp.any(ovf))
            def _fixup():
                def fg(g, carry2):
                    s16 = src_v[pl.ds(g * 16, 16)]
                    d16 = dst_v[pl.ds(g * 16, 16)]
                    for r in range(RPT):
                        v = plsc.load_gather(q_v, [rvecs[r], s16])
                        def fl(l, c3, v=v, d16=d16, r=r):
                            lm = iota16 == l
                            cur = plsc.load_gather(maxs[r], [d16], mask=lm)
                            plsc.store_scatter(maxs[r], [d16],
                                               jnp.maximum(cur, v), mask=lm)
                            return c3
                        lax.fori_loop(0, 16, fl, 0)
                    return carry2
                lax.fori_loop(0, GP, fg, 0)
            return carry
        lax.fori_loop(0, NCH, chunk_body, 0)

        for r in range(RPT):
            pltpu.sync_copy(sums[r], S_hbm.at[rbase + r, :])
            pltpu.sync_copy(maxs[r], M_hbm.at[rbase + r, :])
        if with_cnt:
            pltpu.sync_copy(cnt_v, cnt_hbm.at[pl.ds(nbase, RNODE)])
    return _sc_body


@functools.partial(jax.jit, static_argnames=("with_cnt",))
def _sc_segment(src, dst, qT, with_cnt=True):
    mesh = plsc.VectorSubcoreMesh(core_axis_name="c", subcore_axis_name="s")
    f = pl.kernel(
        _make_sc_body(with_cnt),
        mesh=mesh,
        compiler_params=pltpu.CompilerParams(needs_layout_passes=False),
        out_type=[
            jax.ShapeDtypeStruct((128, NPAD), jnp.float32),
            jax.ShapeDtypeStruct((128, NPAD), jnp.float32),
            jax.ShapeDtypeStruct((NPAD,), jnp.float32),
        ],
        scratch_types=(
            [pltpu.VMEM((RPT, NPAD), jnp.float32)]
            + [pltpu.VMEM((NPAD,), jnp.float32) for _ in range(8)]
            + [
                pltpu.VMEM((RNODE,), jnp.float32),
                pltpu.VMEM((ECH,), jnp.int32),
                pltpu.VMEM((ECH,), jnp.int32),
            ]
        ),
    )
    return f(src, dst, qT)


def kernel(x, edge_index, pre_W1, pre_b1, post_W1, post_b1, lin_W1, lin_b1,
           pre_W2, pre_b2, post_W2, post_b2, lin_W2, lin_b2, W_out, b_out):
    src = edge_index[0]
    dst = edge_index[1]
    xT = jnp.zeros((128, NPAD), jnp.float32).at[:, :N].set(x.T)

    Wd1, Ws1 = pre_W1[:, :D], pre_W1[:, D:]
    Wx1, Wm1, Wmx1, Ws21 = (post_W1[:, 0:128], post_W1[:, 128:256],
                            post_W1[:, 256:384], post_W1[:, 384:512])
    Wd2, Ws2_ = pre_W2[:, :H], pre_W2[:, H:]
    Wx2, Wm2, Wmx2, Ws22 = (post_W2[:, 0:128], post_W2[:, 128:256],
                            post_W2[:, 256:384], post_W2[:, 384:512])

    Wstk1 = jnp.stack([Wd1, Wx1, Wm1, Wmx1, Ws21, lin_W1, Ws2_])
    bstk1 = jnp.stack([pre_b1, post_b1, lin_b1])
    Wstk2 = jnp.stack([Wd2, Wx2, Wm2, Wmx2, Ws22, lin_W2])
    bstk2 = jnp.stack([pre_b2, post_b2, lin_b2])

    qT1 = _tc_pre(xT, Ws1)
    S1, M1, cnt = _sc_segment(src, dst, qT1)
    cnt2d = cnt.reshape(1, NPAD)
    h1T, qT2 = _tc_mid(xT, S1, M1, cnt2d, Wstk1, bstk1)
    S2, M2, _ = _sc_segment(src, dst, qT2, with_cnt=False)
    zT = _tc_fin(h1T, S2, M2, cnt2d, Wstk2, bstk2, W_out)
    return zT[:, :N].T + b_out


# speedup vs baseline: 1.4821x; 1.0021x over previous
import functools

import jax
import jax.numpy as jnp
from jax import lax
from jax.experimental import pallas as pl
from jax.experimental.pallas import tpu as pltpu
from jax.experimental.pallas import tpu_sc as plsc

N = 10000
E = 320000
D = 128
H = 128
C = 64
NPAD = 10240
BN = 1024

ECH = 3200
NCH = E // ECH
GP = ECH // 16
UNROLL = 4
RPT = 4
RNODE = NPAD // 32


def _matmul_T(W, xT):
    return lax.dot_general(W, xT, (((1,), (0,)), ((), ())),
                           preferred_element_type=jnp.float32,
                           precision=lax.Precision.HIGHEST)


def _tc_pre_body(xT_ref, Ws_ref, qT_ref):
    qT_ref[...] = _matmul_T(Ws_ref[...], xT_ref[...])


def _tc_pre(xT, Ws):
    return pl.pallas_call(
        _tc_pre_body,
        grid=(NPAD // BN,),
        in_specs=[
            pl.BlockSpec((128, BN), lambda i: (0, i)),
            pl.BlockSpec((128, 128), lambda i: (0, 0)),
        ],
        out_specs=pl.BlockSpec((128, BN), lambda i: (0, i)),
        out_shape=jax.ShapeDtypeStruct((128, NPAD), jnp.float32),
    )(xT, Ws)


def _tc_mid_body(hT_ref, S_ref, M_ref, cnt_ref, Wstk_ref, bstk_ref, h1T_ref, qT2_ref):
    hT = hT_ref[...]
    S = S_ref[...]
    M = M_ref[...]
    cnt = cnt_ref[...]
    Wd, Wx, Wm, Wmx, Ws2, linW, Wsnext = (Wstk_ref[i] for i in range(7))
    pre_b, post_b, lin_b = (bstk_ref[i][:, None] for i in range(3))
    aT = _matmul_T(Wd, hT) + pre_b
    inv = 1.0 / jnp.maximum(cnt, 1.0)
    meanT = aT + S * inv
    ssumT = cnt * aT + S
    smaxT = jnp.where(cnt > 0, aT + M, 0.0)
    postT = (_matmul_T(Wx, hT) + _matmul_T(Wm, meanT)
             + _matmul_T(Wmx, smaxT) + _matmul_T(Ws2, ssumT) + post_b)
    h1T = jax.nn.relu(_matmul_T(linW, postT) + lin_b)
    h1T_ref[...] = h1T
    qT2_ref[...] = _matmul_T(Wsnext, h1T)


def _tc_mid(hT, S, M, cnt2d, Wstk, bstk):
    return pl.pallas_call(
        _tc_mid_body,
        grid=(NPAD // BN,),
        in_specs=[
            pl.BlockSpec((128, BN), lambda i: (0, i)),
            pl.BlockSpec((128, BN), lambda i: (0, i)),
            pl.BlockSpec((128, BN), lambda i: (0, i)),
            pl.BlockSpec((1, BN), lambda i: (0, i)),
            pl.BlockSpec((7, 128, 128), lambda i: (0, 0, 0)),
            pl.BlockSpec((3, 128), lambda i: (0, 0)),
        ],
        out_specs=[
            pl.BlockSpec((128, BN), lambda i: (0, i)),
            pl.BlockSpec((128, BN), lambda i: (0, i)),
        ],
        out_shape=[
            jax.ShapeDtypeStruct((128, NPAD), jnp.float32),
            jax.ShapeDtypeStruct((128, NPAD), jnp.float32),
        ],
    )(hT, S, M, cnt2d, Wstk, bstk)


def _tc_fin_body(hT_ref, S_ref, M_ref, cnt_ref, Wstk_ref, bstk_ref, Wout_ref, zT_ref):
    hT = hT_ref[...]
    S = S_ref[...]
    M = M_ref[...]
    cnt = cnt_ref[...]
    Wd, Wx, Wm, Wmx, Ws2, linW = (Wstk_ref[i] for i in range(6))
    pre_b, post_b, lin_b = (bstk_ref[i][:, None] for i in range(3))
    aT = _matmul_T(Wd, hT) + pre_b
    inv = 1.0 / jnp.maximum(cnt, 1.0)
    meanT = aT + S * inv
    ssumT = cnt * aT + S
    smaxT = jnp.where(cnt > 0, aT + M, 0.0)
    postT = (_matmul_T(Wx, hT) + _matmul_T(Wm, meanT)
             + _matmul_T(Wmx, smaxT) + _matmul_T(Ws2, ssumT) + post_b)
    h2T = jax.nn.relu(_matmul_T(linW, postT) + lin_b)
    zT_ref[...] = lax.dot_general(Wout_ref[...], h2T, (((1,), (0,)), ((), ())),
                                  preferred_element_type=jnp.float32,
                                  precision=lax.Precision.HIGHEST)


def _tc_fin(hT, S, M, cnt2d, Wstk, bstk, Wout):
    return pl.pallas_call(
        _tc_fin_body,
        grid=(NPAD // BN,),
        in_specs=[
            pl.BlockSpec((128, BN), lambda i: (0, i)),
            pl.BlockSpec((128, BN), lambda i: (0, i)),
            pl.BlockSpec((128, BN), lambda i: (0, i)),
            pl.BlockSpec((1, BN), lambda i: (0, i)),
            pl.BlockSpec((6, 128, 128), lambda i: (0, 0, 0)),
            pl.BlockSpec((3, 128), lambda i: (0, 0)),
            pl.BlockSpec((C, 128), lambda i: (0, 0)),
        ],
        out_specs=pl.BlockSpec((C, BN), lambda i: (0, i)),
        out_shape=jax.ShapeDtypeStruct((C, NPAD), jnp.float32),
    )(hT, S, M, cnt2d, Wstk, bstk, Wout)


def _make_sc_body(with_cnt):
    def _sc_body(src_hbm, dst_hbm, qT_hbm, S_hbm, M_hbm, cnt_hbm,
                 q_v, s0, s1, s2, s3, m0, m1v, m2v, m3, cnt_v, src_v, dst_v):
        sums = [s0, s1, s2, s3]
        maxs = [m0, m1v, m2v, m3]
        c = lax.axis_index("c")
        s = lax.axis_index("s")
        wid = s * 2 + c
        rbase = wid * RPT
        nbase = wid * RNODE

        pltpu.sync_copy(qT_hbm.at[pl.ds(rbase, RPT), :], q_v)

        zeros16 = jnp.zeros((16,), jnp.float32)
        ninf16 = jnp.full((16,), -3.0e38, jnp.float32)
        iota16 = lax.iota(jnp.int32, 16)
        ones16 = jnp.ones((16,), jnp.float32)
        false16 = jnp.zeros((16,), jnp.bool_)
        rvecs = [jnp.full((16,), r, jnp.int32) for r in range(RPT)]

        def zbody(i, carry):
            for r in range(RPT):
                sums[r][pl.ds(i * 16, 16)] = zeros16
                maxs[r][pl.ds(i * 16, 16)] = ninf16
            return carry
        lax.fori_loop(0, NPAD // 16, zbody, 0)

        if with_cnt:
            def cbody(i, carry):
                cnt_v[pl.ds(i * 16, 16)] = zeros16
                return carry
            lax.fori_loop(0, RNODE // 16, cbody, 0)

        def do_group(g, ovf):
            s16 = src_v[pl.ds(g * 16, 16)]
            d16 = dst_v[pl.ds(g * 16, 16)]
            occ, _last = plsc.scan_count(d16)
            m1 = occ == 1
            m2 = occ == 2
            ovf = ovf | (occ >= 3)
            if with_cnt:
                mine = (d16 >= nbase) & (d16 < nbase + RNODE)
                plsc.addupdate_scatter(cnt_v, [d16 - nbase], ones16, mask=mine)
            vs = []
            for r in range(RPT):
                v = plsc.load_gather(q_v, [rvecs[r], s16])
                vs.append(v)
                plsc.addupdate_scatter(sums[r], [d16], v)
                cur = plsc.load_gather(maxs[r], [d16])
                plsc.store_scatter(maxs[r], [d16],
                                   jnp.maximum(cur, v), mask=m1)

            @pl.when(jnp.any(occ >= 2))
            def _round2():
                for r in range(RPT):
                    cur2 = plsc.load_gather(maxs[r], [d16], mask=m2)
                    plsc.store_scatter(maxs[r], [d16],
                                       jnp.maximum(cur2, vs[r]), mask=m2)
            return ovf

        def chunk_body(ch, carry):
            pltpu.sync_copy(src_hbm.at[pl.ds(ch * ECH, ECH)], src_v)
            pltpu.sync_copy(dst_hbm.at[pl.ds(ch * ECH, ECH)], dst_v)

            def group_body(i, ovf):
                for u in range(UNROLL):
                    ovf = do_group(i * UNROLL + u, ovf)
                return ovf
            ovf = lax.fori_loop(0, GP // UNROLL, group_body, false16)

            @pl.when(jnp.any(ovf))
            def _fixup():
                def fg(g, carry2):
                    s16 = src_v[pl.ds(g * 16, 16)]
                    d16 = dst_v[pl.ds(g * 16, 16)]
                    for r in range(RPT):
                        v = plsc.load_gather(q_v, [rvecs[r], s16])
                        def fl(l, c3, v=v, d16=d16, r=r):
                            lm = iota16 == l
                            cur = plsc.load_gather(maxs[r], [d16], mask=lm)
                            plsc.store_scatter(maxs[r], [d16],
                                               jnp.maximum(cur, v), mask=lm)
                            return c3
                        lax.fori_loop(0, 16, fl, 0)
                    return carry2
                lax.fori_loop(0, GP, fg, 0)
            return carry
        lax.fori_loop(0, NCH, chunk_body, 0)

        for r in range(RPT):
            pltpu.sync_copy(sums[r], S_hbm.at[rbase + r, :])
            pltpu.sync_copy(maxs[r], M_hbm.at[rbase + r, :])
        if with_cnt:
            pltpu.sync_copy(cnt_v, cnt_hbm.at[pl.ds(nbase, RNODE)])
    return _sc_body


@functools.partial(jax.jit, static_argnames=("with_cnt",))
def _sc_segment(src, dst, qT, with_cnt=True):
    mesh = plsc.VectorSubcoreMesh(core_axis_name="c", subcore_axis_name="s")
    f = pl.kernel(
        _make_sc_body(with_cnt),
        mesh=mesh,
        compiler_params=pltpu.CompilerParams(needs_layout_passes=False),
        out_type=[
            jax.ShapeDtypeStruct((128, NPAD), jnp.float32),
            jax.ShapeDtypeStruct((128, NPAD), jnp.float32),
            jax.ShapeDtypeStruct((NPAD,), jnp.float32),
        ],
        scratch_types=(
            [pltpu.VMEM((RPT, NPAD), jnp.float32)]
            + [pltpu.VMEM((NPAD,), jnp.float32) for _ in range(8)]
            + [
                pltpu.VMEM((RNODE,), jnp.float32),
                pltpu.VMEM((ECH,), jnp.int32),
                pltpu.VMEM((ECH,), jnp.int32),
            ]
        ),
    )
    return f(src, dst, qT)


def kernel(x, edge_index, pre_W1, pre_b1, post_W1, post_b1, lin_W1, lin_b1,
           pre_W2, pre_b2, post_W2, post_b2, lin_W2, lin_b2, W_out, b_out):
    src = edge_index[0]
    dst = edge_index[1]
    xT = jnp.zeros((128, NPAD), jnp.float32).at[:, :N].set(x.T)

    Wd1, Ws1 = pre_W1[:, :D], pre_W1[:, D:]
    Wx1, Wm1, Wmx1, Ws21 = (post_W1[:, 0:128], post_W1[:, 128:256],
                            post_W1[:, 256:384], post_W1[:, 384:512])
    Wd2, Ws2_ = pre_W2[:, :H], pre_W2[:, H:]
    Wx2, Wm2, Wmx2, Ws22 = (post_W2[:, 0:128], post_W2[:, 128:256],
                            post_W2[:, 256:384], post_W2[:, 384:512])

    Wstk1 = jnp.stack([Wd1, Wx1, Wm1, Wmx1, Ws21, lin_W1, Ws2_])
    bstk1 = jnp.stack([pre_b1, post_b1, lin_b1])
    Wstk2 = jnp.stack([Wd2, Wx2, Wm2, Wmx2, Ws22, lin_W2])
    bstk2 = jnp.stack([pre_b2, post_b2, lin_b2])

    qT1 = _tc_pre(xT, Ws1)
    S1, M1, cnt = _sc_segment(src, dst, qT1)
    cnt2d = cnt.reshape(1, NPAD)
    h1T, qT2 = _tc_mid(xT, S1, M1, cnt2d, Wstk1, bstk1)
    S2, M2, _ = _sc_segment(src, dst, qT2, with_cnt=False)
    zT = _tc_fin(h1T, S2, M2, cnt2d, Wstk2, bstk2, W_out)
    return zT[:, :N].T + b_out


# speedup vs baseline: 1.4832x; 1.0007x over previous
import functools

import jax
import jax.numpy as jnp
from jax import lax
from jax.experimental import pallas as pl
from jax.experimental.pallas import tpu as pltpu
from jax.experimental.pallas import tpu_sc as plsc

N = 10000
E = 320000
D = 128
H = 128
C = 64
NPAD = 10240
BN = 1024

ECH = 3200
NCH = E // ECH
GP = ECH // 16
UNROLL = 8
RPT = 4
RNODE = NPAD // 32


def _matmul_T(W, xT):
    return lax.dot_general(W, xT, (((1,), (0,)), ((), ())),
                           preferred_element_type=jnp.float32,
                           precision=lax.Precision.HIGHEST)


def _tc_pre_body(xT_ref, Ws_ref, qT_ref):
    qT_ref[...] = _matmul_T(Ws_ref[...], xT_ref[...])


def _tc_pre(xT, Ws):
    return pl.pallas_call(
        _tc_pre_body,
        grid=(NPAD // BN,),
        in_specs=[
            pl.BlockSpec((128, BN), lambda i: (0, i)),
            pl.BlockSpec((128, 128), lambda i: (0, 0)),
        ],
        out_specs=pl.BlockSpec((128, BN), lambda i: (0, i)),
        out_shape=jax.ShapeDtypeStruct((128, NPAD), jnp.float32),
    )(xT, Ws)


def _tc_mid_body(hT_ref, S_ref, M_ref, cnt_ref, Wstk_ref, bstk_ref, h1T_ref, qT2_ref):
    hT = hT_ref[...]
    S = S_ref[...]
    M = M_ref[...]
    cnt = cnt_ref[...]
    Wd, Wx, Wm, Wmx, Ws2, linW, Wsnext = (Wstk_ref[i] for i in range(7))
    pre_b, post_b, lin_b = (bstk_ref[i][:, None] for i in range(3))
    aT = _matmul_T(Wd, hT) + pre_b
    inv = 1.0 / jnp.maximum(cnt, 1.0)
    meanT = aT + S * inv
    ssumT = cnt * aT + S
    smaxT = jnp.where(cnt > 0, aT + M, 0.0)
    postT = (_matmul_T(Wx, hT) + _matmul_T(Wm, meanT)
             + _matmul_T(Wmx, smaxT) + _matmul_T(Ws2, ssumT) + post_b)
    h1T = jax.nn.relu(_matmul_T(linW, postT) + lin_b)
    h1T_ref[...] = h1T
    qT2_ref[...] = _matmul_T(Wsnext, h1T)


def _tc_mid(hT, S, M, cnt2d, Wstk, bstk):
    return pl.pallas_call(
        _tc_mid_body,
        grid=(NPAD // BN,),
        in_specs=[
            pl.BlockSpec((128, BN), lambda i: (0, i)),
            pl.BlockSpec((128, BN), lambda i: (0, i)),
            pl.BlockSpec((128, BN), lambda i: (0, i)),
            pl.BlockSpec((1, BN), lambda i: (0, i)),
            pl.BlockSpec((7, 128, 128), lambda i: (0, 0, 0)),
            pl.BlockSpec((3, 128), lambda i: (0, 0)),
        ],
        out_specs=[
            pl.BlockSpec((128, BN), lambda i: (0, i)),
            pl.BlockSpec((128, BN), lambda i: (0, i)),
        ],
        out_shape=[
            jax.ShapeDtypeStruct((128, NPAD), jnp.float32),
            jax.ShapeDtypeStruct((128, NPAD), jnp.float32),
        ],
    )(hT, S, M, cnt2d, Wstk, bstk)


def _tc_fin_body(hT_ref, S_ref, M_ref, cnt_ref, Wstk_ref, bstk_ref, Wout_ref, zT_ref):
    hT = hT_ref[...]
    S = S_ref[...]
    M = M_ref[...]
    cnt = cnt_ref[...]
    Wd, Wx, Wm, Wmx, Ws2, linW = (Wstk_ref[i] for i in range(6))
    pre_b, post_b, lin_b = (bstk_ref[i][:, None] for i in range(3))
    aT = _matmul_T(Wd, hT) + pre_b
    inv = 1.0 / jnp.maximum(cnt, 1.0)
    meanT = aT + S * inv
    ssumT = cnt * aT + S
    smaxT = jnp.where(cnt > 0, aT + M, 0.0)
    postT = (_matmul_T(Wx, hT) + _matmul_T(Wm, meanT)
             + _matmul_T(Wmx, smaxT) + _matmul_T(Ws2, ssumT) + post_b)
    h2T = jax.nn.relu(_matmul_T(linW, postT) + lin_b)
    zT_ref[...] = lax.dot_general(Wout_ref[...], h2T, (((1,), (0,)), ((), ())),
                                  preferred_element_type=jnp.float32,
                                  precision=lax.Precision.HIGHEST)


def _tc_fin(hT, S, M, cnt2d, Wstk, bstk, Wout):
    return pl.pallas_call(
        _tc_fin_body,
        grid=(NPAD // BN,),
        in_specs=[
            pl.BlockSpec((128, BN), lambda i: (0, i)),
            pl.BlockSpec((128, BN), lambda i: (0, i)),
            pl.BlockSpec((128, BN), lambda i: (0, i)),
            pl.BlockSpec((1, BN), lambda i: (0, i)),
            pl.BlockSpec((6, 128, 128), lambda i: (0, 0, 0)),
            pl.BlockSpec((3, 128), lambda i: (0, 0)),
            pl.BlockSpec((C, 128), lambda i: (0, 0)),
        ],
        out_specs=pl.BlockSpec((C, BN), lambda i: (0, i)),
        out_shape=jax.ShapeDtypeStruct((C, NPAD), jnp.float32),
    )(hT, S, M, cnt2d, Wstk, bstk, Wout)


def _make_sc_body(with_cnt):
    def _sc_body(src_hbm, dst_hbm, qT_hbm, S_hbm, M_hbm, cnt_hbm,
                 q_v, s0, s1, s2, s3, m0, m1v, m2v, m3, cnt_v, src_v, dst_v):
        sums = [s0, s1, s2, s3]
        maxs = [m0, m1v, m2v, m3]
        c = lax.axis_index("c")
        s = lax.axis_index("s")
        wid = s * 2 + c
        rbase = wid * RPT
        nbase = wid * RNODE

        pltpu.sync_copy(qT_hbm.at[pl.ds(rbase, RPT), :], q_v)

        zeros16 = jnp.zeros((16,), jnp.float32)
        ninf16 = jnp.full((16,), -3.0e38, jnp.float32)
        iota16 = lax.iota(jnp.int32, 16)
        ones16 = jnp.ones((16,), jnp.float32)
        false16 = jnp.zeros((16,), jnp.bool_)
        rvecs = [jnp.full((16,), r, jnp.int32) for r in range(RPT)]

        def zbody(i, carry):
            for r in range(RPT):
                sums[r][pl.ds(i * 16, 16)] = zeros16
                maxs[r][pl.ds(i * 16, 16)] = ninf16
            return carry
        lax.fori_loop(0, NPAD // 16, zbody, 0)

        if with_cnt:
            def cbody(i, carry):
                cnt_v[pl.ds(i * 16, 16)] = zeros16
                return carry
            lax.fori_loop(0, RNODE // 16, cbody, 0)

        def do_group(g, ovf):
            s16 = src_v[pl.ds(g * 16, 16)]
            d16 = dst_v[pl.ds(g * 16, 16)]
            occ, _last = plsc.scan_count(d16)
            m1 = occ == 1
            m2 = occ == 2
            ovf = ovf | (occ >= 3)
            if with_cnt:
                mine = (d16 >= nbase) & (d16 < nbase + RNODE)
                plsc.addupdate_scatter(cnt_v, [d16 - nbase], ones16, mask=mine)
            vs = []
            for r in range(RPT):
                v = plsc.load_gather(q_v, [rvecs[r], s16])
                vs.append(v)
                plsc.addupdate_scatter(sums[r], [d16], v)
                cur = plsc.load_gather(maxs[r], [d16])
                plsc.store_scatter(maxs[r], [d16],
                                   jnp.maximum(cur, v), mask=m1)

            @pl.when(jnp.any(occ >= 2))
            def _round2():
                for r in range(RPT):
                    cur2 = plsc.load_gather(maxs[r], [d16], mask=m2)
                    plsc.store_scatter(maxs[r], [d16],
                                       jnp.maximum(cur2, vs[r]), mask=m2)
            return ovf

        def chunk_body(ch, carry):
            pltpu.sync_copy(src_hbm.at[pl.ds(ch * ECH, ECH)], src_v)
            pltpu.sync_copy(dst_hbm.at[pl.ds(ch * ECH, ECH)], dst_v)

            def group_body(i, ovf):
                for u in range(UNROLL):
                    ovf = do_group(i * UNROLL + u, ovf)
                return ovf
            ovf = lax.fori_loop(0, GP // UNROLL, group_body, false16)

            @pl.when(jnp.any(ovf))
            def _fixup():
                def fg(g, carry2):
                    s16 = src_v[pl.ds(g * 16, 16)]
                    d16 = dst_v[pl.ds(g * 16, 16)]
                    for r in range(RPT):
                        v = plsc.load_gather(q_v, [rvecs[r], s16])
                        def fl(l, c3, v=v, d16=d16, r=r):
                            lm = iota16 == l
                            cur = plsc.load_gather(maxs[r], [d16], mask=lm)
                            plsc.store_scatter(maxs[r], [d16],
                                               jnp.maximum(cur, v), mask=lm)
                            return c3
                        lax.fori_loop(0, 16, fl, 0)
                    return carry2
                lax.fori_loop(0, GP, fg, 0)
            return carry
        lax.fori_loop(0, NCH, chunk_body, 0)

        for r in range(RPT):
            pltpu.sync_copy(sums[r], S_hbm.at[rbase + r, :])
            pltpu.sync_copy(maxs[r], M_hbm.at[rbase + r, :])
        if with_cnt:
            pltpu.sync_copy(cnt_v, cnt_hbm.at[pl.ds(nbase, RNODE)])
    return _sc_body


@functools.partial(jax.jit, static_argnames=("with_cnt",))
def _sc_segment(src, dst, qT, with_cnt=True):
    mesh = plsc.VectorSubcoreMesh(core_axis_name="c", subcore_axis_name="s")
    f = pl.kernel(
        _make_sc_body(with_cnt),
        mesh=mesh,
        compiler_params=pltpu.CompilerParams(needs_layout_passes=False),
        out_type=[
            jax.ShapeDtypeStruct((128, NPAD), jnp.float32),
            jax.ShapeDtypeStruct((128, NPAD), jnp.float32),
            jax.ShapeDtypeStruct((NPAD,), jnp.float32),
        ],
        scratch_types=(
            [pltpu.VMEM((RPT, NPAD), jnp.float32)]
            + [pltpu.VMEM((NPAD,), jnp.float32) for _ in range(8)]
            + [
                pltpu.VMEM((RNODE,), jnp.float32),
                pltpu.VMEM((ECH,), jnp.int32),
                pltpu.VMEM((ECH,), jnp.int32),
            ]
        ),
    )
    return f(src, dst, qT)


def kernel(x, edge_index, pre_W1, pre_b1, post_W1, post_b1, lin_W1, lin_b1,
           pre_W2, pre_b2, post_W2, post_b2, lin_W2, lin_b2, W_out, b_out):
    src = edge_index[0]
    dst = edge_index[1]
    xT = jnp.zeros((128, NPAD), jnp.float32).at[:, :N].set(x.T)

    Wd1, Ws1 = pre_W1[:, :D], pre_W1[:, D:]
    Wx1, Wm1, Wmx1, Ws21 = (post_W1[:, 0:128], post_W1[:, 128:256],
                            post_W1[:, 256:384], post_W1[:, 384:512])
    Wd2, Ws2_ = pre_W2[:, :H], pre_W2[:, H:]
    Wx2, Wm2, Wmx2, Ws22 = (post_W2[:, 0:128], post_W2[:, 128:256],
                            post_W2[:, 256:384], post_W2[:, 384:512])

    Wstk1 = jnp.stack([Wd1, Wx1, Wm1, Wmx1, Ws21, lin_W1, Ws2_])
    bstk1 = jnp.stack([pre_b1, post_b1, lin_b1])
    Wstk2 = jnp.stack([Wd2, Wx2, Wm2, Wmx2, Ws22, lin_W2])
    bstk2 = jnp.stack([pre_b2, post_b2, lin_b2])

    qT1 = _tc_pre(xT, Ws1)
    S1, M1, cnt = _sc_segment(src, dst, qT1)
    cnt2d = cnt.reshape(1, NPAD)
    h1T, qT2 = _tc_mid(xT, S1, M1, cnt2d, Wstk1, bstk1)
    S2, M2, _ = _sc_segment(src, dst, qT2, with_cnt=False)
    zT = _tc_fin(h1T, S2, M2, cnt2d, Wstk2, bstk2, W_out)
    return zT[:, :N].T + b_out
